# baseline (speedup 1.0000x reference)
import functools
import math

import jax
import jax.numpy as jnp
from jax.experimental import pallas as pl
from jax.experimental.pallas import tpu as pltpu
from jax.experimental.pallas import tpu_sc as plsc

B, T, D, H, E, TED = 2, 2048, 1024, 2048, 8, 512
N = B * T
G = 2 * E
A = 4 * N
TB = 256
NT = N // TB
BLK = 256
P_CAP = A + G * BLK
NBLK = P_CAP // BLK
W = 128
NWC = N // W
IW = A // W
S = D // 4

_NEG = -1e30
_INV_SQRT2 = 0.7071067811865476


def _gate_kernel(x_ref, g0_ref, b0_ref, gw0_ref, gb0_ref,
                 g1_ref, b1_ref, gw1_ref, gb1_ref,
                 xf_ref, gidx_ref, gval_ref):
    xb = x_ref[...]
    m = jnp.mean(xb, axis=1, keepdims=True)
    xc = xb - m
    v = jnp.mean(xc * xc, axis=1, keepdims=True)
    xhat = xc * jax.lax.rsqrt(v + 1e-5)
    eidx = jax.lax.broadcasted_iota(jnp.int32, (TB, E), 1)
    lane8 = jax.lax.broadcasted_iota(jnp.int32, (TB, 8), 1)
    gidx = jnp.zeros((TB, 8), jnp.int32)
    gval = jnp.zeros((TB, 8), jnp.float32)
    for br, (g_r, b_r, gw_r, gb_r) in enumerate(
            ((g0_ref, b0_ref, gw0_ref, gb0_ref),
             (g1_ref, b1_ref, gw1_ref, gb1_ref))):
        xf = xhat * g_r[...] + b_r[...]
        for c in range(4):
            xf_ref[c, br] = xf[:, c * S:(c + 1) * S]
        logits = jnp.dot(xf, gw_r[...], preferred_element_type=jnp.float32)
        logits = logits + gb_r[...]
        lmax = jnp.max(logits, axis=1, keepdims=True)
        ex = jnp.exp(logits - lmax)
        probs = ex / jnp.sum(ex, axis=1, keepdims=True)
        m1 = jnp.max(probs, axis=1, keepdims=True)
        i1 = jnp.min(jnp.where(probs >= m1, eidx, E), axis=1, keepdims=True)
        p2 = jnp.where(eidx == i1, _NEG, probs)
        m2 = jnp.max(p2, axis=1, keepdims=True)
        i2 = jnp.min(jnp.where(p2 >= m2, eidx, E), axis=1, keepdims=True)
        c2 = 2 * br
        gidx = gidx + jnp.where(lane8 == c2, i1 + br * E, 0)
        gidx = gidx + jnp.where(lane8 == c2 + 1, i2 + br * E, 0)
        gval = gval + jnp.where(lane8 == c2, m1, 0.0)
        gval = gval + jnp.where(lane8 == c2 + 1, m2, 0.0)
    gidx_ref[...] = gidx
    gval_ref[...] = gval


def _gmm_kernel(gblk_ref, bval_ref, x_ref, w1_ref, b1_ref, w2_ref, b2_ref,
                y_ref):
    k = pl.program_id(0)

    @pl.when(bval_ref[k] == 1)
    def _():
        h = b1_ref[0].astype(jnp.float32) * jnp.ones((BLK, 1), jnp.float32)
        for c in range(4):
            h = h + jnp.dot(x_ref[c].astype(jnp.bfloat16),
                            w1_ref[0, c * S:(c + 1) * S, :],
                            preferred_element_type=jnp.float32)
        h = 0.5 * h * (1.0 + jax.lax.erf(h * _INV_SQRT2))
        hb = h.astype(jnp.bfloat16)
        for c in range(4):
            y = jnp.dot(hb, w2_ref[0, :, c * S:(c + 1) * S],
                        preferred_element_type=jnp.float32)
            y_ref[c] = y + b2_ref[0, :, c * S:(c + 1) * S]


def _rank_kernel(gidx_ref, rank_ref, cnt_ref, carry_ref):
    s = pl.program_id(0)
    c = s // NT

    @pl.when(s == 0)
    def _():
        carry_ref[...] = jnp.zeros((8, G), jnp.float32)

    lane8 = jax.lax.broadcasted_iota(jnp.int32, (TB, 8), 1)
    gcol = jnp.sum(jnp.where(lane8 == c, gidx_ref[...], 0), axis=1,
                   keepdims=True)
    laneg = jax.lax.broadcasted_iota(jnp.int32, (TB, G), 1)
    oh = (gcol == laneg).astype(jnp.float32)
    r_i = jax.lax.broadcasted_iota(jnp.int32, (TB, TB), 0)
    c_i = jax.lax.broadcasted_iota(jnp.int32, (TB, TB), 1)
    tri = (r_i > c_i).astype(jnp.float32)
    excl = jnp.dot(tri, oh, preferred_element_type=jnp.float32)
    r = excl + carry_ref[0:1]
    rv = jnp.sum(jnp.where(oh > 0, r, 0.0), axis=1, keepdims=True)
    rank_ref[0] = (rv * jnp.ones((1, 8), jnp.float32)).astype(jnp.int32)
    carry_ref[0:1] = carry_ref[0:1] + jnp.sum(oh, axis=0, keepdims=True)

    @pl.when(s == 4 * NT - 1)
    def _():
        cnt_ref[...] = carry_ref[...].astype(jnp.int32)


def _emb_kernel(emb_ref, ew_ref, eb_ref, sc_ref):
    e = emb_ref[...]
    e = e * jax.nn.sigmoid(e)
    sc_ref[...] = jnp.dot(e, ew_ref[...],
                          preferred_element_type=jnp.float32) + eb_ref[...]


def _style_kernel(x_ref, yg_ref, gval_ref,
                  sc_ref, sh_ref, sg_ref, sb_ref, ow_ref, ob_ref, out_ref):
    v0 = gval_ref[:, 0:1]
    v1 = gval_ref[:, 1:2]
    v2 = gval_ref[:, 2:3]
    v3 = gval_ref[:, 3:4]
    o = []
    tot = jnp.zeros((TB, 1), jnp.float32)
    for c in range(4):
        oc = (v0 * yg_ref[c, 0] + v1 * yg_ref[c, 1]
              + v2 * yg_ref[c, 2] + v3 * yg_ref[c, 3]) * 0.5
        o.append(oc)
        tot = tot + jnp.sum(oc, axis=1, keepdims=True)
    m = tot / D
    var = jnp.zeros((TB, 1), jnp.float32)
    for c in range(4):
        d = o[c] - m
        var = var + jnp.sum(d * d, axis=1, keepdims=True)
    rstd = jax.lax.rsqrt(var / D + 1e-5)
    acc = ob_ref[...] * jnp.ones((TB, 1), jnp.float32)
    for c in range(4):
        sl = slice(c * S, (c + 1) * S)
        hh = (o[c] - m) * rstd * sg_ref[:, sl] + sb_ref[:, sl]
        hh = hh * (1.0 + sc_ref[0, :, sl]) + sh_ref[0, :, sl]
        hh = hh * jax.nn.sigmoid(hh)
        acc = acc + jnp.dot(hh.astype(jnp.bfloat16), ow_ref[c],
                            preferred_element_type=jnp.float32)
    out_ref[...] = x_ref[...] + acc


def _sc_mesh():
    return plsc.VectorSubcoreMesh(core_axis_name="core",
                                  subcore_axis_name="subcore")


def _dispatch_scatter(xfs, phc):
    @pl.kernel(out_type=jax.ShapeDtypeStruct((4 * P_CAP, S), jnp.float32),
               mesh=_sc_mesh())
    def sc_kernel(xf_hbm, pos_hbm, xs_hbm):
        def body(x_vmem, i_vmem):
            pltpu.sync_copy(x_vmem, xs_hbm.at[i_vmem.at[0]])

        def src_map(i):
            s, j = i // IW, i % IW
            return (s * (2 * NWC) + (j // NWC) // 2 * NWC + j % NWC, 0)

        pltpu.emit_pipeline(
            body,
            grid=(4 * IW,),
            in_specs=[
                pl.BlockSpec((W, S), index_map=src_map),
                pl.BlockSpec((1, W), index_map=lambda i: (0, i)),
            ],
            out_specs=[],
            core_axis_name=("core", "subcore"),
            dimension_semantics=(pltpu.PARALLEL,),
        )(xf_hbm, pos_hbm)

    return sc_kernel(xfs, phc)


def _combine_gather(ys, phc):
    @pl.kernel(out_type=jax.ShapeDtypeStruct((4 * A, S), jnp.float32),
               mesh=_sc_mesh())
    def sc_kernel(y_hbm, pos_hbm, yg_hbm):
        def body(i_vmem, o_vmem):
            pltpu.sync_copy(y_hbm.at[i_vmem.at[0]], o_vmem)

        pltpu.emit_pipeline(
            body,
            grid=(4 * IW,),
            in_specs=[pl.BlockSpec((1, W), index_map=lambda i: (0, i))],
            out_specs=[pl.BlockSpec((W, S), index_map=lambda i: (i, 0))],
            core_axis_name=("core", "subcore"),
            dimension_semantics=(pltpu.PARALLEL,),
        )(pos_hbm, yg_hbm)

    return sc_kernel(ys, phc)


def kernel(x, emb, ln_g0, ln_b0, gate_w0, gate_b0, w1_0, b1_0, w2_0, b2_0,
           ln_g1, ln_b1, gate_w1, gate_b1, w1_1, b1_1, w2_1, b2_1,
           emb_w, emb_b, sb_g, sb_b, out_w, out_b):
    x2 = x.reshape(N, D)
    row = lambda a: a.reshape(1, -1)

    xf, gidx, gval = pl.pallas_call(
        _gate_kernel,
        grid=(NT,),
        in_specs=[
            pl.BlockSpec((TB, D), lambda t: (t, 0)),
            pl.BlockSpec((1, D), lambda t: (0, 0)),
            pl.BlockSpec((1, D), lambda t: (0, 0)),
            pl.BlockSpec((D, E), lambda t: (0, 0)),
            pl.BlockSpec((1, E), lambda t: (0, 0)),
            pl.BlockSpec((1, D), lambda t: (0, 0)),
            pl.BlockSpec((1, D), lambda t: (0, 0)),
            pl.BlockSpec((D, E), lambda t: (0, 0)),
            pl.BlockSpec((1, E), lambda t: (0, 0)),
        ],
        out_specs=[
            pl.BlockSpec((4, 2, TB, S), lambda t: (0, 0, t, 0)),
            pl.BlockSpec((TB, 8), lambda t: (t, 0)),
            pl.BlockSpec((TB, 8), lambda t: (t, 0)),
        ],
        out_shape=[
            jax.ShapeDtypeStruct((4, 2, N, S), jnp.float32),
            jax.ShapeDtypeStruct((N, 8), jnp.int32),
            jax.ShapeDtypeStruct((N, 8), jnp.float32),
        ],
    )(x2, row(ln_g0), row(ln_b0), gate_w0, row(gate_b0),
      row(ln_g1), row(ln_b1), gate_w1, row(gate_b1))

    rank_out, cnts = pl.pallas_call(
        _rank_kernel,
        grid=(4 * NT,),
        in_specs=[pl.BlockSpec((TB, 8), lambda s: (s % NT, 0))],
        out_specs=[
            pl.BlockSpec((1, TB, 8), lambda s: (s, 0, 0)),
            pl.BlockSpec((8, G), lambda s: (0, 0)),
        ],
        out_shape=[
            jax.ShapeDtypeStruct((4 * NT, TB, 8), jnp.int32),
            jax.ShapeDtypeStruct((8, G), jnp.int32),
        ],
        scratch_shapes=[pltpu.VMEM((8, G), jnp.float32)],
        compiler_params=pltpu.CompilerParams(
            dimension_semantics=("arbitrary",)),
    )(gidx)
    rank = rank_out[:, :, 0].reshape(A)
    counts = cnts[0]
    ga = gidx[:, :4].T.reshape(-1)
    onehot = (ga[:, None] == jnp.arange(G, dtype=jnp.int32)[None, :])
    padded = ((counts + BLK - 1) // BLK) * BLK
    poffs = jnp.concatenate([jnp.zeros((1,), jnp.int32),
                             jnp.cumsum(padded)[:-1].astype(jnp.int32)])
    total_padded = poffs[-1] + padded[-1]
    pos = jnp.sum(jnp.where(onehot, poffs[None, :], 0), axis=1) + rank
    pos = pos.astype(jnp.int32)
    phc = (pos[None, :]
           + (jnp.arange(4, dtype=jnp.int32) * P_CAP)[:, None]).reshape(1,
                                                                        4 * A)
    block_starts = jnp.arange(NBLK, dtype=jnp.int32) * BLK
    gblk = jnp.sum((block_starts[:, None] >= poffs[None, :]).astype(jnp.int32),
                   axis=1) - 1
    bval = (block_starts < total_padded).astype(jnp.int32)

    xs = _dispatch_scatter(xf.reshape(8 * N, S), phc).reshape(4, P_CAP, S)

    w1s = jnp.concatenate([w1_0, w1_1], axis=0).astype(jnp.bfloat16)
    w2s = jnp.concatenate([w2_0, w2_1], axis=0).astype(jnp.bfloat16)
    b1s = jnp.concatenate([b1_0, b1_1], axis=0).reshape(G, 1, H)
    b2s = jnp.concatenate([b2_0, b2_1], axis=0).reshape(G, 1, D)

    ys = pl.pallas_call(
        _gmm_kernel,
        grid_spec=pltpu.PrefetchScalarGridSpec(
            num_scalar_prefetch=2,
            grid=(NBLK,),
            in_specs=[
                pl.BlockSpec((4, BLK, S), lambda k, gr, vr: (0, k, 0)),
                pl.BlockSpec((1, D, H), lambda k, gr, vr: (gr[k], 0, 0)),
                pl.BlockSpec((1, 1, H), lambda k, gr, vr: (gr[k], 0, 0)),
                pl.BlockSpec((1, H, D), lambda k, gr, vr: (gr[k], 0, 0)),
                pl.BlockSpec((1, 1, D), lambda k, gr, vr: (gr[k], 0, 0)),
            ],
            out_specs=pl.BlockSpec((4, BLK, S), lambda k, gr, vr: (0, k, 0)),
        ),
        out_shape=jax.ShapeDtypeStruct((4, P_CAP, S), jnp.float32),
        compiler_params=pltpu.CompilerParams(
            dimension_semantics=("arbitrary",)),
    )(gblk, bval, xs, w1s, b1s, w2s, b2s)

    yg = _combine_gather(ys.reshape(4 * P_CAP, S), phc).reshape(4, 4, N, S)

    sc = pl.pallas_call(
        _emb_kernel,
        in_specs=[
            pl.BlockSpec((B, TED), lambda: (0, 0)),
            pl.BlockSpec((TED, 2 * D), lambda: (0, 0)),
            pl.BlockSpec((1, 2 * D), lambda: (0, 0)),
        ],
        out_specs=pl.BlockSpec((B, 2 * D), lambda: (0, 0)),
        out_shape=jax.ShapeDtypeStruct((B, 2 * D), jnp.float32),
    )(emb, emb_w, row(emb_b))
    scale = sc[:, :D].reshape(B, 1, D)
    shift = sc[:, D:].reshape(B, 1, D)

    TPB = T // TB
    out = pl.pallas_call(
        _style_kernel,
        grid=(NT,),
        in_specs=[
            pl.BlockSpec((TB, D), lambda t: (t, 0)),
            pl.BlockSpec((4, 4, TB, S), lambda t: (0, 0, t, 0)),
            pl.BlockSpec((TB, 8), lambda t: (t, 0)),
            pl.BlockSpec((1, 1, D), lambda t: (t // TPB, 0, 0)),
            pl.BlockSpec((1, 1, D), lambda t: (t // TPB, 0, 0)),
            pl.BlockSpec((1, D), lambda t: (0, 0)),
            pl.BlockSpec((1, D), lambda t: (0, 0)),
            pl.BlockSpec((4, S, D), lambda t: (0, 0, 0)),
            pl.BlockSpec((1, D), lambda t: (0, 0)),
        ],
        out_specs=pl.BlockSpec((TB, D), lambda t: (t, 0)),
        out_shape=jax.ShapeDtypeStruct((N, D), jnp.float32),
    )(x2, yg, gval, scale, shift, row(sb_g), row(sb_b),
      out_w.reshape(4, S, D).astype(jnp.bfloat16), row(out_b))

    return out.reshape(x.shape)


# speedup vs baseline: 1.0714x; 1.0714x over previous
import functools
import math

import jax
import jax.numpy as jnp
from jax.experimental import pallas as pl
from jax.experimental.pallas import tpu as pltpu
from jax.experimental.pallas import tpu_sc as plsc

B, T, D, H, E, TED = 2, 2048, 1024, 2048, 8, 512
N = B * T
G = 2 * E
A = 4 * N
TB = 256
NT = N // TB
BLK = 256
P_CAP = A + G * BLK
NBLK = P_CAP // BLK
W = 128
NWC = N // W
IW = A // W
S = D // 4

_NEG = -1e30
_INV_SQRT2 = 0.7071067811865476


def _gate_kernel(x_ref, g0_ref, b0_ref, gw0_ref, gb0_ref,
                 g1_ref, b1_ref, gw1_ref, gb1_ref,
                 xf_ref, gidx_ref, gval_ref):
    xb = x_ref[...]
    m = jnp.mean(xb, axis=1, keepdims=True)
    xc = xb - m
    v = jnp.mean(xc * xc, axis=1, keepdims=True)
    xhat = xc * jax.lax.rsqrt(v + 1e-5)
    eidx = jax.lax.broadcasted_iota(jnp.int32, (TB, E), 1)
    lane8 = jax.lax.broadcasted_iota(jnp.int32, (TB, 8), 1)
    gidx = jnp.zeros((TB, 8), jnp.int32)
    gval = jnp.zeros((TB, 8), jnp.float32)
    for br, (g_r, b_r, gw_r, gb_r) in enumerate(
            ((g0_ref, b0_ref, gw0_ref, gb0_ref),
             (g1_ref, b1_ref, gw1_ref, gb1_ref))):
        xf = xhat * g_r[...] + b_r[...]
        for c in range(4):
            xf_ref[c, br] = xf[:, c * S:(c + 1) * S]
        logits = jnp.dot(xf, gw_r[...], preferred_element_type=jnp.float32)
        logits = logits + gb_r[...]
        lmax = jnp.max(logits, axis=1, keepdims=True)
        ex = jnp.exp(logits - lmax)
        probs = ex / jnp.sum(ex, axis=1, keepdims=True)
        m1 = jnp.max(probs, axis=1, keepdims=True)
        i1 = jnp.min(jnp.where(probs >= m1, eidx, E), axis=1, keepdims=True)
        p2 = jnp.where(eidx == i1, _NEG, probs)
        m2 = jnp.max(p2, axis=1, keepdims=True)
        i2 = jnp.min(jnp.where(p2 >= m2, eidx, E), axis=1, keepdims=True)
        c2 = 2 * br
        gidx = gidx + jnp.where(lane8 == c2, i1 + br * E, 0)
        gidx = gidx + jnp.where(lane8 == c2 + 1, i2 + br * E, 0)
        gval = gval + jnp.where(lane8 == c2, m1, 0.0)
        gval = gval + jnp.where(lane8 == c2 + 1, m2, 0.0)
    gidx_ref[...] = gidx
    gval_ref[...] = gval


def _gmm_kernel(gblk_ref, bval_ref, x_ref, w1_ref, b1_ref, w2_ref, b2_ref,
                y_ref):
    k = pl.program_id(0)

    @pl.when(bval_ref[k] == 1)
    def _():
        h = b1_ref[0].astype(jnp.float32) * jnp.ones((BLK, 1), jnp.float32)
        for c in range(4):
            h = h + jnp.dot(x_ref[c].astype(jnp.bfloat16),
                            w1_ref[0, c * S:(c + 1) * S, :],
                            preferred_element_type=jnp.float32)
        h = 0.5 * h * (1.0 + jax.lax.erf(h * _INV_SQRT2))
        hb = h.astype(jnp.bfloat16)
        for c in range(4):
            y = jnp.dot(hb, w2_ref[0, :, c * S:(c + 1) * S],
                        preferred_element_type=jnp.float32)
            y_ref[c] = y + b2_ref[0, :, c * S:(c + 1) * S]


def _emb_kernel(emb_ref, ew_ref, eb_ref, sc_ref):
    e = emb_ref[...]
    e = e * jax.nn.sigmoid(e)
    sc_ref[...] = jnp.dot(e, ew_ref[...],
                          preferred_element_type=jnp.float32) + eb_ref[...]


def _style_kernel(x_ref, yg_ref, gval_ref,
                  sc_ref, sh_ref, sg_ref, sb_ref, ow_ref, ob_ref, out_ref):
    v0 = gval_ref[:, 0:1]
    v1 = gval_ref[:, 1:2]
    v2 = gval_ref[:, 2:3]
    v3 = gval_ref[:, 3:4]
    o = []
    tot = jnp.zeros((TB, 1), jnp.float32)
    for c in range(4):
        oc = (v0 * yg_ref[c, 0] + v1 * yg_ref[c, 1]
              + v2 * yg_ref[c, 2] + v3 * yg_ref[c, 3]) * 0.5
        o.append(oc)
        tot = tot + jnp.sum(oc, axis=1, keepdims=True)
    m = tot / D
    var = jnp.zeros((TB, 1), jnp.float32)
    for c in range(4):
        d = o[c] - m
        var = var + jnp.sum(d * d, axis=1, keepdims=True)
    rstd = jax.lax.rsqrt(var / D + 1e-5)
    acc = ob_ref[...] * jnp.ones((TB, 1), jnp.float32)
    for c in range(4):
        sl = slice(c * S, (c + 1) * S)
        hh = (o[c] - m) * rstd * sg_ref[:, sl] + sb_ref[:, sl]
        hh = hh * (1.0 + sc_ref[0, :, sl]) + sh_ref[0, :, sl]
        hh = hh * jax.nn.sigmoid(hh)
        acc = acc + jnp.dot(hh.astype(jnp.bfloat16), ow_ref[c],
                            preferred_element_type=jnp.float32)
    out_ref[...] = x_ref[...] + acc


def _sc_mesh():
    return plsc.VectorSubcoreMesh(core_axis_name="core",
                                  subcore_axis_name="subcore")


def _dispatch_scatter(xfs, phc):
    @pl.kernel(out_type=jax.ShapeDtypeStruct((4 * P_CAP, S), jnp.float32),
               mesh=_sc_mesh())
    def sc_kernel(xf_hbm, pos_hbm, xs_hbm):
        def body(x_vmem, i_vmem):
            pltpu.sync_copy(x_vmem, xs_hbm.at[i_vmem.at[0]])

        def src_map(i):
            s, j = i // IW, i % IW
            return (s * (2 * NWC) + (j // NWC) // 2 * NWC + j % NWC, 0)

        pltpu.emit_pipeline(
            body,
            grid=(4 * IW,),
            in_specs=[
                pl.BlockSpec((W, S), index_map=src_map),
                pl.BlockSpec((1, W), index_map=lambda i: (0, i)),
            ],
            out_specs=[],
            core_axis_name=("core", "subcore"),
            dimension_semantics=(pltpu.PARALLEL,),
        )(xf_hbm, pos_hbm)

    return sc_kernel(xfs, phc)


def _combine_gather(ys, phc):
    @pl.kernel(out_type=jax.ShapeDtypeStruct((4 * A, S), jnp.float32),
               mesh=_sc_mesh())
    def sc_kernel(y_hbm, pos_hbm, yg_hbm):
        def body(i_vmem, o_vmem):
            pltpu.sync_copy(y_hbm.at[i_vmem.at[0]], o_vmem)

        pltpu.emit_pipeline(
            body,
            grid=(4 * IW,),
            in_specs=[pl.BlockSpec((1, W), index_map=lambda i: (0, i))],
            out_specs=[pl.BlockSpec((W, S), index_map=lambda i: (i, 0))],
            core_axis_name=("core", "subcore"),
            dimension_semantics=(pltpu.PARALLEL,),
        )(pos_hbm, yg_hbm)

    return sc_kernel(ys, phc)


def kernel(x, emb, ln_g0, ln_b0, gate_w0, gate_b0, w1_0, b1_0, w2_0, b2_0,
           ln_g1, ln_b1, gate_w1, gate_b1, w1_1, b1_1, w2_1, b2_1,
           emb_w, emb_b, sb_g, sb_b, out_w, out_b):
    x2 = x.reshape(N, D)
    row = lambda a: a.reshape(1, -1)

    xf, gidx, gval = pl.pallas_call(
        _gate_kernel,
        grid=(NT,),
        in_specs=[
            pl.BlockSpec((TB, D), lambda t: (t, 0)),
            pl.BlockSpec((1, D), lambda t: (0, 0)),
            pl.BlockSpec((1, D), lambda t: (0, 0)),
            pl.BlockSpec((D, E), lambda t: (0, 0)),
            pl.BlockSpec((1, E), lambda t: (0, 0)),
            pl.BlockSpec((1, D), lambda t: (0, 0)),
            pl.BlockSpec((1, D), lambda t: (0, 0)),
            pl.BlockSpec((D, E), lambda t: (0, 0)),
            pl.BlockSpec((1, E), lambda t: (0, 0)),
        ],
        out_specs=[
            pl.BlockSpec((4, 2, TB, S), lambda t: (0, 0, t, 0)),
            pl.BlockSpec((TB, 8), lambda t: (t, 0)),
            pl.BlockSpec((TB, 8), lambda t: (t, 0)),
        ],
        out_shape=[
            jax.ShapeDtypeStruct((4, 2, N, S), jnp.float32),
            jax.ShapeDtypeStruct((N, 8), jnp.int32),
            jax.ShapeDtypeStruct((N, 8), jnp.float32),
        ],
    )(x2, row(ln_g0), row(ln_b0), gate_w0, row(gate_b0),
      row(ln_g1), row(ln_b1), gate_w1, row(gate_b1))

    ga = gidx[:, :4].T.reshape(-1)
    onehot = (ga[:, None] == jnp.arange(G, dtype=jnp.int32)[None, :])
    onehot_i = onehot.astype(jnp.int32)
    cum = jnp.cumsum(onehot_i, axis=0)
    counts = cum[-1]
    rank = jnp.sum((cum - 1) * onehot_i, axis=1)
    padded = ((counts + BLK - 1) // BLK) * BLK
    poffs = jnp.concatenate([jnp.zeros((1,), jnp.int32),
                             jnp.cumsum(padded)[:-1].astype(jnp.int32)])
    total_padded = poffs[-1] + padded[-1]
    pos = jnp.sum(jnp.where(onehot, poffs[None, :], 0), axis=1) + rank
    pos = pos.astype(jnp.int32)
    phc = (pos[None, :]
           + (jnp.arange(4, dtype=jnp.int32) * P_CAP)[:, None]).reshape(1,
                                                                        4 * A)
    block_starts = jnp.arange(NBLK, dtype=jnp.int32) * BLK
    gblk = jnp.sum((block_starts[:, None] >= poffs[None, :]).astype(jnp.int32),
                   axis=1) - 1
    bval = (block_starts < total_padded).astype(jnp.int32)

    xs = _dispatch_scatter(xf.reshape(8 * N, S), phc).reshape(4, P_CAP, S)

    w1s = jnp.concatenate([w1_0, w1_1], axis=0).astype(jnp.bfloat16)
    w2s = jnp.concatenate([w2_0, w2_1], axis=0).astype(jnp.bfloat16)
    b1s = jnp.concatenate([b1_0, b1_1], axis=0).reshape(G, 1, H)
    b2s = jnp.concatenate([b2_0, b2_1], axis=0).reshape(G, 1, D)

    ys = pl.pallas_call(
        _gmm_kernel,
        grid_spec=pltpu.PrefetchScalarGridSpec(
            num_scalar_prefetch=2,
            grid=(NBLK,),
            in_specs=[
                pl.BlockSpec((4, BLK, S), lambda k, gr, vr: (0, k, 0)),
                pl.BlockSpec((1, D, H), lambda k, gr, vr: (gr[k], 0, 0)),
                pl.BlockSpec((1, 1, H), lambda k, gr, vr: (gr[k], 0, 0)),
                pl.BlockSpec((1, H, D), lambda k, gr, vr: (gr[k], 0, 0)),
                pl.BlockSpec((1, 1, D), lambda k, gr, vr: (gr[k], 0, 0)),
            ],
            out_specs=pl.BlockSpec((4, BLK, S), lambda k, gr, vr: (0, k, 0)),
        ),
        out_shape=jax.ShapeDtypeStruct((4, P_CAP, S), jnp.float32),
        compiler_params=pltpu.CompilerParams(
            dimension_semantics=("arbitrary",)),
    )(gblk, bval, xs, w1s, b1s, w2s, b2s)

    yg = _combine_gather(ys.reshape(4 * P_CAP, S), phc).reshape(4, 4, N, S)

    sc = pl.pallas_call(
        _emb_kernel,
        in_specs=[
            pl.BlockSpec((B, TED), lambda: (0, 0)),
            pl.BlockSpec((TED, 2 * D), lambda: (0, 0)),
            pl.BlockSpec((1, 2 * D), lambda: (0, 0)),
        ],
        out_specs=pl.BlockSpec((B, 2 * D), lambda: (0, 0)),
        out_shape=jax.ShapeDtypeStruct((B, 2 * D), jnp.float32),
    )(emb, emb_w, row(emb_b))
    scale = sc[:, :D].reshape(B, 1, D)
    shift = sc[:, D:].reshape(B, 1, D)

    TPB = T // TB
    out = pl.pallas_call(
        _style_kernel,
        grid=(NT,),
        in_specs=[
            pl.BlockSpec((TB, D), lambda t: (t, 0)),
            pl.BlockSpec((4, 4, TB, S), lambda t: (0, 0, t, 0)),
            pl.BlockSpec((TB, 8), lambda t: (t, 0)),
            pl.BlockSpec((1, 1, D), lambda t: (t // TPB, 0, 0)),
            pl.BlockSpec((1, 1, D), lambda t: (t // TPB, 0, 0)),
            pl.BlockSpec((1, D), lambda t: (0, 0)),
            pl.BlockSpec((1, D), lambda t: (0, 0)),
            pl.BlockSpec((4, S, D), lambda t: (0, 0, 0)),
            pl.BlockSpec((1, D), lambda t: (0, 0)),
        ],
        out_specs=pl.BlockSpec((TB, D), lambda t: (t, 0)),
        out_shape=jax.ShapeDtypeStruct((N, D), jnp.float32),
    )(x2, yg, gval, scale, shift, row(sb_g), row(sb_b),
      out_w.reshape(4, S, D).astype(jnp.bfloat16), row(out_b))

    return out.reshape(x.shape)
